# baseline (speedup 1.0000x reference)
import functools

import jax
import jax.numpy as jnp
from jax import lax
from jax.experimental import pallas as pl
from jax.experimental.pallas import tpu as pltpu
from jax.experimental.pallas import tpu_sc as plsc

_N_GRID = 100000
_N_MESH = 10000
_N_EDGE = 160000
_D_EDGE = 16
_D = 128
_GRID_DIM = 2
_K = 4
_L = 6

_NC, _NS, _LANES = 2, 16, 16
_NW = _NC * _NS
_CHUNK = 128

_E_PAD = 163840
_ACC_ROWS = 10240
_DUMMY_ROW = _ACC_ROWS - 1
_G2M_PAD = 40960
_M2G_PAD = 409600

_BG = 2000
_BM = 2000


def _silu(x):
    return x * (1.0 / (1.0 + jnp.exp(-x)))


def _grid_enc_body(x, w0, b0, w1, b1, o):
    h = _silu(x[...] @ w0[...] + b0[...])
    o[...] = h @ w1[...] + b1[...]


_grid_enc = pl.pallas_call(
    _grid_enc_body,
    grid=(_N_GRID // _BG,),
    in_specs=[
        pl.BlockSpec((_BG, _GRID_DIM), lambda i: (i, 0)),
        pl.BlockSpec((_GRID_DIM, _D), lambda i: (0, 0)),
        pl.BlockSpec((1, _D), lambda i: (0, 0)),
        pl.BlockSpec((_D, _D), lambda i: (0, 0)),
        pl.BlockSpec((1, _D), lambda i: (0, 0)),
    ],
    out_specs=pl.BlockSpec((_BG, _D), lambda i: (i, 0)),
    out_shape=jax.ShapeDtypeStruct((_N_GRID, _D), jnp.float32),
)


def _tdeg_body(t0, t1, o_t, o_d):
    s = t0[...] + t1[...]
    o_t[...] = s[:, :_D_EDGE]
    o_d[...] = s[:, _D_EDGE:_D_EDGE + 1]


_tdeg_combine = pl.pallas_call(
    _tdeg_body,
    grid=(_N_MESH // _BM,),
    in_specs=[
        pl.BlockSpec((_BM, _D), lambda i: (i, 0)),
        pl.BlockSpec((_BM, _D), lambda i: (i, 0)),
    ],
    out_specs=[
        pl.BlockSpec((_BM, _D_EDGE), lambda i: (i, 0)),
        pl.BlockSpec((_BM, 1), lambda i: (i, 0)),
    ],
    out_shape=[
        jax.ShapeDtypeStruct((_N_MESH, _D_EDGE), jnp.float32),
        jax.ShapeDtypeStruct((_N_MESH, 1), jnp.float32),
    ],
)


def _pool_mlp_body(nb, w, extra, w0a, w0b, b0, w1, b1, o):
    x = nb[...]
    ww = w[...]
    pooled = x[:, 0 * _D:1 * _D] * ww[:, 0:1]
    pooled += x[:, 1 * _D:2 * _D] * ww[:, 1:2]
    pooled += x[:, 2 * _D:3 * _D] * ww[:, 2:3]
    pooled += x[:, 3 * _D:4 * _D] * ww[:, 3:4]
    h = _silu(pooled @ w0a[...] + extra[...] @ w0b[...] + b0[...])
    o[...] = h @ w1[...] + b1[...]


_mesh_enc = pl.pallas_call(
    _pool_mlp_body,
    grid=(_N_MESH // _BM,),
    in_specs=[
        pl.BlockSpec((_BM, _K * _D), lambda i: (i, 0)),
        pl.BlockSpec((_BM, _K), lambda i: (i, 0)),
        pl.BlockSpec((_BM, _D), lambda i: (i, 0)),
        pl.BlockSpec((_D, _D), lambda i: (0, 0)),
        pl.BlockSpec((_D, _D), lambda i: (0, 0)),
        pl.BlockSpec((1, _D), lambda i: (0, 0)),
        pl.BlockSpec((_D, _D), lambda i: (0, 0)),
        pl.BlockSpec((1, _D), lambda i: (0, 0)),
    ],
    out_specs=pl.BlockSpec((_BM, _D), lambda i: (i, 0)),
    out_shape=jax.ShapeDtypeStruct((_N_MESH, _D), jnp.float32),
)


def _dec_body(nb, w, w0, b0, w1, b1, o):
    x = nb[...]
    ww = w[...]
    pooled = x[:, 0 * _D:1 * _D] * ww[:, 0:1]
    pooled += x[:, 1 * _D:2 * _D] * ww[:, 1:2]
    pooled += x[:, 2 * _D:3 * _D] * ww[:, 2:3]
    pooled += x[:, 3 * _D:4 * _D] * ww[:, 3:4]
    h = _silu(pooled @ w0[...] + b0[...])
    o[...] = h @ w1[...] + b1[...]


_decoder = pl.pallas_call(
    _dec_body,
    grid=(_N_GRID // _BG,),
    in_specs=[
        pl.BlockSpec((_BG, _K * _D), lambda i: (i, 0)),
        pl.BlockSpec((_BG, _K), lambda i: (i, 0)),
        pl.BlockSpec((_D, _D), lambda i: (0, 0)),
        pl.BlockSpec((1, _D), lambda i: (0, 0)),
        pl.BlockSpec((_D, _GRID_DIM), lambda i: (0, 0)),
        pl.BlockSpec((1, _GRID_DIM), lambda i: (0, 0)),
    ],
    out_specs=pl.BlockSpec((_BG, _GRID_DIM), lambda i: (i, 0)),
    out_shape=jax.ShapeDtypeStruct((_N_GRID, _GRID_DIM), jnp.float32),
)


def _layer_a_body(n, t, dg, ws, wr, we, be, o_ps, o_sp):
    nn = n[...]
    o_ps[...] = nn @ ws[...]
    o_sp[...] = dg[...] * (nn @ wr[...] + be[...]) + t[...] @ we[...]


def _make_layer_a(dt):
    return pl.pallas_call(
        _layer_a_body,
        grid=(_N_MESH // _BM,),
        in_specs=[
            pl.BlockSpec((_BM, _D), lambda i: (i, 0)),
            pl.BlockSpec((_BM, dt), lambda i: (i, 0)),
            pl.BlockSpec((_BM, 1), lambda i: (i, 0)),
            pl.BlockSpec((_D, _D), lambda i: (0, 0)),
            pl.BlockSpec((_D, _D), lambda i: (0, 0)),
            pl.BlockSpec((dt, _D), lambda i: (0, 0)),
            pl.BlockSpec((1, _D), lambda i: (0, 0)),
        ],
        out_specs=[
            pl.BlockSpec((_BM, _D), lambda i: (i, 0)),
            pl.BlockSpec((_BM, _D), lambda i: (i, 0)),
        ],
        out_shape=[
            jax.ShapeDtypeStruct((_N_MESH, _D), jnp.float32),
            jax.ShapeDtypeStruct((_N_MESH, _D), jnp.float32),
        ],
    )


_layer_a_first = _make_layer_a(_D_EDGE)
_layer_a_rest = _make_layer_a(_D)


def _layer_c_body(n, sp, g0, g1, w0a, w0b, b0, w1, b1, o_n, o_s):
    nn = n[...]
    s = sp[...] + g0[...] + g1[...]
    h = _silu(nn @ w0a[...] + s @ w0b[...] + b0[...])
    o_n[...] = nn + h @ w1[...] + b1[...]
    o_s[...] = s


_layer_c = pl.pallas_call(
    _layer_c_body,
    grid=(_N_MESH // _BM,),
    in_specs=[
        pl.BlockSpec((_BM, _D), lambda i: (i, 0)),
        pl.BlockSpec((_BM, _D), lambda i: (i, 0)),
        pl.BlockSpec((_BM, _D), lambda i: (i, 0)),
        pl.BlockSpec((_BM, _D), lambda i: (i, 0)),
        pl.BlockSpec((_D, _D), lambda i: (0, 0)),
        pl.BlockSpec((_D, _D), lambda i: (0, 0)),
        pl.BlockSpec((1, _D), lambda i: (0, 0)),
        pl.BlockSpec((_D, _D), lambda i: (0, 0)),
        pl.BlockSpec((1, _D), lambda i: (0, 0)),
    ],
    out_specs=[
        pl.BlockSpec((_BM, _D), lambda i: (i, 0)),
        pl.BlockSpec((_BM, _D), lambda i: (i, 0)),
    ],
    out_shape=[
        jax.ShapeDtypeStruct((_N_MESH, _D), jnp.float32),
        jax.ShapeDtypeStruct((_N_MESH, _D), jnp.float32),
    ],
)


def _make_sc_gather(n_rows, d, b_pad, krow, nbuf):
    chunk = krow * _CHUNK
    per_w = b_pad // _NW
    n_chunks = per_w // chunk
    n_grp = n_chunks // nbuf
    idx_rows = n_chunks + nbuf
    mesh = plsc.VectorSubcoreMesh(core_axis_name="c", subcore_axis_name="s")

    scratch = [pltpu.VMEM((idx_rows, chunk), jnp.int32)]
    scratch += [pltpu.VMEM((chunk, d), jnp.float32) for _ in range(nbuf)]
    scratch += [pltpu.SemaphoreType.DMA for _ in range(2 * nbuf)]

    @functools.partial(
        pl.kernel,
        out_type=jax.ShapeDtypeStruct((b_pad, d), jnp.float32),
        mesh=mesh,
        scratch_types=scratch,
    )
    def gather_kernel(table_hbm, idx_hbm, out_hbm, idx_v, *bufs):
        rows = bufs[:nbuf]
        gsems = bufs[nbuf:2 * nbuf]
        osems = bufs[2 * nbuf:]
        wid = lax.axis_index("s") * _NC + lax.axis_index("c")
        base = wid * per_w
        pltpu.sync_copy(idx_hbm.at[wid], idx_v)

        def idx_at(j):
            return idx_v.at[j]

        def start_gather(b, j):
            pltpu.async_copy(table_hbm.at[idx_at(j)], rows[b], gsems[b])

        def drain_gather(b):
            pltpu.make_async_copy(table_hbm.at[idx_at(0)], rows[b],
                                  gsems[b]).wait()

        for b in range(nbuf):
            start_gather(b, b)

        def grp_body(g, carry):
            j0 = g * nbuf
            for b in range(nbuf):
                drain_gather(b)
                pltpu.async_copy(
                    rows[b],
                    out_hbm.at[pl.ds(base + (j0 + b) * chunk, chunk)],
                    osems[b])
            for b in range(nbuf):
                pltpu.make_async_copy(
                    rows[b],
                    out_hbm.at[pl.ds(base, chunk)],
                    osems[b]).wait()
                start_gather(b, j0 + nbuf + b)
            return carry

        lax.fori_loop(0, n_grp, grp_body, 0)
        for b in range(nbuf):
            drain_gather(b)

    return gather_kernel


def _make_sc_scatter_add(d, table_mode):
    nbuf = 2
    per_w = _E_PAD // _NW
    n_chunks = per_w // _CHUNK
    n_grp = n_chunks // nbuf
    idx_rows = n_chunks + nbuf
    rows_per_tile = _ACC_ROWS // _NS
    mesh = plsc.VectorSubcoreMesh(core_axis_name="c", subcore_axis_name="s")

    scratch = [
        pltpu.VMEM((idx_rows, _CHUNK), jnp.int32),
        pltpu.VMEM((idx_rows, _CHUNK), jnp.int32),
        pltpu.VMEM((16, d), jnp.float32),
        pltpu.VMEM_SHARED((_ACC_ROWS, d), jnp.float32),
        pltpu.SemaphoreType.DMA,
    ]
    scratch += [pltpu.VMEM((_CHUNK, d), jnp.float32) for _ in range(nbuf)]
    scratch += [pltpu.SemaphoreType.DMA for _ in range(2 * nbuf)]

    @functools.partial(
        pl.kernel,
        out_type=jax.ShapeDtypeStruct((_NC, _ACC_ROWS, d), jnp.float32),
        mesh=mesh,
        scratch_types=scratch,
    )
    def scatter_kernel(src_hbm, sidx_hbm, ridx_hbm, out_hbm,
                       sidx_v, ridx_v, zbuf, acc, zsem, *bufs):
        rows = bufs[:nbuf]
        gsems = bufs[nbuf:2 * nbuf]
        ssems = bufs[2 * nbuf:]
        cid = lax.axis_index("c")
        sid = lax.axis_index("s")
        wid = sid * _NC + cid
        row0 = sid * rows_per_tile

        zv = jnp.zeros((_LANES,), jnp.float32)

        def zb_body(r, carry):
            for c in range(d // _LANES):
                zbuf[r, pl.ds(c * _LANES, _LANES)] = zv
            return carry

        lax.fori_loop(0, 16, zb_body, 0)
        zds = [pltpu.async_copy(zbuf, acc.at[pl.ds(row0 + t * 16, 16)], zsem)
               for t in range(rows_per_tile // 16)]
        if table_mode:
            pltpu.sync_copy(sidx_hbm.at[wid], sidx_v)
        pltpu.sync_copy(ridx_hbm.at[wid], ridx_v)
        for dcp in zds:
            dcp.wait()
        plsc.subcore_barrier()

        def start_gather(b, j):
            if table_mode:
                pltpu.async_copy(src_hbm.at[sidx_v.at[j]], rows[b],
                                 gsems[b])
            else:
                pltpu.async_copy(
                    src_hbm.at[pl.ds((wid * n_chunks + j) * _CHUNK, _CHUNK)],
                    rows[b], gsems[b])

        def drain_gather(b):
            if table_mode:
                pltpu.make_async_copy(src_hbm.at[sidx_v.at[0]], rows[b],
                                      gsems[b]).wait()
            else:
                pltpu.make_async_copy(
                    src_hbm.at[pl.ds(0, _CHUNK)], rows[b], gsems[b]).wait()

        for b in range(nbuf):
            start_gather(b, b)

        def grp_body(g, carry):
            j0 = g * nbuf
            for b in range(nbuf):
                drain_gather(b)
                pltpu.async_copy(rows[b], acc.at[ridx_v.at[j0 + b]],
                                 ssems[b], add=True)
            for b in range(nbuf):
                pltpu.make_async_copy(rows[b], acc.at[ridx_v.at[0]],
                                      ssems[b]).wait()
                start_gather(b, j0 + nbuf + b)
            return carry

        lax.fori_loop(0, n_grp, grp_body, 0)
        for b in range(nbuf):
            drain_gather(b)
        plsc.subcore_barrier()
        pltpu.sync_copy(acc.at[pl.ds(row0, rows_per_tile)],
                        out_hbm.at[cid, pl.ds(row0, rows_per_tile)])

    return scatter_kernel


_G2M_NBUF = 2
_M2G_NBUF = 4
_g2m_gather = _make_sc_gather(_N_GRID, _D, _G2M_PAD, krow=1, nbuf=_G2M_NBUF)
_m2g_gather = _make_sc_gather(_N_MESH, _D, _M2G_PAD, krow=1, nbuf=_M2G_NBUF)
_edge_scatter = _make_sc_scatter_add(_D, table_mode=True)
_init_scatter = _make_sc_scatter_add(_D, table_mode=False)
_SC_NBUF = 2


def _pack_idx(flat, b_pad, krow, nbuf):
    chunk = krow * _CHUNK
    arr = flat.reshape(_NW, b_pad // _NW)
    arr = jnp.concatenate(
        [arr, jnp.zeros((_NW, nbuf * chunk), jnp.int32)], axis=1)
    return arr.reshape(_NW, -1, chunk)


def kernel(grid_input, mesh_nodes, mesh_edges, mesh_senders, mesh_receivers,
           g2m_indices, g2m_weights, m2g_indices, m2g_weights,
           grid_enc_W0, grid_enc_b0, grid_enc_W1, grid_enc_b1,
           mesh_enc_W0, mesh_enc_b0, mesh_enc_W1, mesh_enc_b1,
           edge_W_first, edge_b_first, edge_W_rest, edge_b_rest,
           node_W0, node_b0, node_W1, node_b1,
           dec_W0, dec_b0, dec_W1, dec_b1):
    i32 = jnp.int32
    senders = mesh_senders.astype(i32)
    receivers = mesh_receivers.astype(i32)
    e_pad = _E_PAD - _N_EDGE
    sidx = _pack_idx(jnp.concatenate([senders, jnp.zeros((e_pad,), i32)]),
                     _E_PAD, 1, _SC_NBUF)
    ridx = _pack_idx(
        jnp.concatenate([receivers, jnp.full((e_pad,), _DUMMY_ROW, i32)]),
        _E_PAD, 1, _SC_NBUF)

    grid_feat = _grid_enc(grid_input, grid_enc_W0,
                          grid_enc_b0.reshape(1, _D),
                          grid_enc_W1, grid_enc_b1.reshape(1, _D))

    g2m_flat = _pack_idx(jnp.concatenate([
        g2m_indices.reshape(-1).astype(i32),
        jnp.zeros((_G2M_PAD - _N_MESH * _K,), i32)]),
        _G2M_PAD, 1, _G2M_NBUF)
    g2m_rows = _g2m_gather(grid_feat, g2m_flat)
    g2m_rows = g2m_rows[:_N_MESH * _K].reshape(_N_MESH, _K * _D)

    payload = jnp.concatenate([
        mesh_edges,
        jnp.ones((_N_EDGE, 1), jnp.float32),
        jnp.zeros((_N_EDGE, _D - _D_EDGE - 1), jnp.float32)], axis=1)
    payload = jnp.concatenate(
        [payload,
         jnp.zeros((e_pad + _SC_NBUF * _CHUNK, _D), jnp.float32)], axis=0)
    tparts = _init_scatter(payload, sidx, ridx)
    t_init, deg = _tdeg_combine(tparts[0, :_N_MESH], tparts[1, :_N_MESH])

    nodes = _mesh_enc(g2m_rows, g2m_weights, mesh_nodes,
                      mesh_enc_W0[:_D], mesh_enc_W0[_D:],
                      mesh_enc_b0.reshape(1, _D),
                      mesh_enc_W1, mesh_enc_b1.reshape(1, _D))

    t_prev = t_init
    for l in range(_L):
        if l == 0:
            ws, wr = edge_W_first[:_D], edge_W_first[_D:2 * _D]
            we, be = edge_W_first[2 * _D:], edge_b_first
            layer_a = _layer_a_first
        else:
            ew = edge_W_rest[l - 1]
            ws, wr, we = ew[:_D], ew[_D:2 * _D], ew[2 * _D:]
            be = edge_b_rest[l - 1]
            layer_a = _layer_a_rest
        p_s, s_pre = layer_a(nodes, t_prev, deg, ws, wr, we,
                             be.reshape(1, _D))
        gparts = _edge_scatter(p_s, sidx, ridx)
        nodes, t_prev = _layer_c(nodes, s_pre,
                                 gparts[0, :_N_MESH], gparts[1, :_N_MESH],
                                 node_W0[l, :_D], node_W0[l, _D:],
                                 node_b0[l].reshape(1, _D),
                                 node_W1[l], node_b1[l].reshape(1, _D))

    m2g_flat = _pack_idx(jnp.concatenate([
        m2g_indices.reshape(-1).astype(i32),
        jnp.zeros((_M2G_PAD - _N_GRID * _K,), i32)]),
        _M2G_PAD, 1, _M2G_NBUF)
    m2g_rows = _m2g_gather(nodes, m2g_flat)
    m2g_rows = m2g_rows[:_N_GRID * _K].reshape(_N_GRID, _K * _D)

    return _decoder(m2g_rows, m2g_weights, dec_W0,
                    dec_b0.reshape(1, _D), dec_W1,
                    dec_b1.reshape(1, _GRID_DIM))


# speedup vs baseline: 1.7602x; 1.7602x over previous
import functools

import jax
import jax.numpy as jnp
from jax import lax
from jax.experimental import pallas as pl
from jax.experimental.pallas import tpu as pltpu
from jax.experimental.pallas import tpu_sc as plsc

_N_GRID = 100000
_N_MESH = 10000
_N_EDGE = 160000
_D_EDGE = 16
_D = 128
_GRID_DIM = 2
_K = 4
_L = 6

_NC, _NS, _LANES = 2, 16, 16
_NW = _NC * _NS
_CHUNK = 128

_E_PAD = 163840
_ACC_ROWS = 10240
_DUMMY_ROW = _ACC_ROWS - 1
_G2M_PAD = 40960
_M2G_PAD = 409600

_BG = 2000
_BM = 2000


def _silu(x):
    return x * (1.0 / (1.0 + jnp.exp(-x)))


def _grid_enc_body(x, w0, b0, w1, b1, o):
    h = _silu(x[...] @ w0[...] + b0[...])
    o[...] = h @ w1[...] + b1[...]


_grid_enc = pl.pallas_call(
    _grid_enc_body,
    grid=(_N_GRID // _BG,),
    in_specs=[
        pl.BlockSpec((_BG, _GRID_DIM), lambda i: (i, 0)),
        pl.BlockSpec((_GRID_DIM, _D), lambda i: (0, 0)),
        pl.BlockSpec((1, _D), lambda i: (0, 0)),
        pl.BlockSpec((_D, _D), lambda i: (0, 0)),
        pl.BlockSpec((1, _D), lambda i: (0, 0)),
    ],
    out_specs=pl.BlockSpec((_BG, _D), lambda i: (i, 0)),
    out_shape=jax.ShapeDtypeStruct((_N_GRID, _D), jnp.float32),
)


def _tdeg_body(t0, t1, o_t, o_d):
    s = t0[...] + t1[...]
    o_t[...] = s[:, :_D_EDGE]
    o_d[...] = s[:, _D_EDGE:_D_EDGE + 1]


_tdeg_combine = pl.pallas_call(
    _tdeg_body,
    grid=(_N_MESH // _BM,),
    in_specs=[
        pl.BlockSpec((_BM, _D), lambda i: (i, 0)),
        pl.BlockSpec((_BM, _D), lambda i: (i, 0)),
    ],
    out_specs=[
        pl.BlockSpec((_BM, _D_EDGE), lambda i: (i, 0)),
        pl.BlockSpec((_BM, 1), lambda i: (i, 0)),
    ],
    out_shape=[
        jax.ShapeDtypeStruct((_N_MESH, _D_EDGE), jnp.float32),
        jax.ShapeDtypeStruct((_N_MESH, 1), jnp.float32),
    ],
)


def _pool_mlp_body(nb, w, extra, w0a, w0b, b0, w1, b1, o):
    x = nb[...]
    ww = w[...]
    pooled = x[:, 0 * _D:1 * _D] * ww[:, 0:1]
    pooled += x[:, 1 * _D:2 * _D] * ww[:, 1:2]
    pooled += x[:, 2 * _D:3 * _D] * ww[:, 2:3]
    pooled += x[:, 3 * _D:4 * _D] * ww[:, 3:4]
    h = _silu(pooled @ w0a[...] + extra[...] @ w0b[...] + b0[...])
    o[...] = h @ w1[...] + b1[...]


_mesh_enc = pl.pallas_call(
    _pool_mlp_body,
    grid=(_N_MESH // _BM,),
    in_specs=[
        pl.BlockSpec((_BM, _K * _D), lambda i: (i, 0)),
        pl.BlockSpec((_BM, _K), lambda i: (i, 0)),
        pl.BlockSpec((_BM, _D), lambda i: (i, 0)),
        pl.BlockSpec((_D, _D), lambda i: (0, 0)),
        pl.BlockSpec((_D, _D), lambda i: (0, 0)),
        pl.BlockSpec((1, _D), lambda i: (0, 0)),
        pl.BlockSpec((_D, _D), lambda i: (0, 0)),
        pl.BlockSpec((1, _D), lambda i: (0, 0)),
    ],
    out_specs=pl.BlockSpec((_BM, _D), lambda i: (i, 0)),
    out_shape=jax.ShapeDtypeStruct((_N_MESH, _D), jnp.float32),
)


def _dec_body(nb, w, w0, b0, w1, b1, o):
    x = nb[...]
    ww = w[...]
    pooled = x[:, 0 * _D:1 * _D] * ww[:, 0:1]
    pooled += x[:, 1 * _D:2 * _D] * ww[:, 1:2]
    pooled += x[:, 2 * _D:3 * _D] * ww[:, 2:3]
    pooled += x[:, 3 * _D:4 * _D] * ww[:, 3:4]
    h = _silu(pooled @ w0[...] + b0[...])
    o[...] = h @ w1[...] + b1[...]


_decoder = pl.pallas_call(
    _dec_body,
    grid=(_N_GRID // _BG,),
    in_specs=[
        pl.BlockSpec((_BG, _K * _D), lambda i: (i, 0)),
        pl.BlockSpec((_BG, _K), lambda i: (i, 0)),
        pl.BlockSpec((_D, _D), lambda i: (0, 0)),
        pl.BlockSpec((1, _D), lambda i: (0, 0)),
        pl.BlockSpec((_D, _GRID_DIM), lambda i: (0, 0)),
        pl.BlockSpec((1, _GRID_DIM), lambda i: (0, 0)),
    ],
    out_specs=pl.BlockSpec((_BG, _GRID_DIM), lambda i: (i, 0)),
    out_shape=jax.ShapeDtypeStruct((_N_GRID, _GRID_DIM), jnp.float32),
)


def _layer_a_body(n, t, dg, ws, wr, we, be, o_ps, o_sp):
    nn = n[...]
    o_ps[...] = nn @ ws[...]
    o_sp[...] = dg[...] * (nn @ wr[...] + be[...]) + t[...] @ we[...]


def _make_layer_a(dt):
    return pl.pallas_call(
        _layer_a_body,
        grid=(_N_MESH // _BM,),
        in_specs=[
            pl.BlockSpec((_BM, _D), lambda i: (i, 0)),
            pl.BlockSpec((_BM, dt), lambda i: (i, 0)),
            pl.BlockSpec((_BM, 1), lambda i: (i, 0)),
            pl.BlockSpec((_D, _D), lambda i: (0, 0)),
            pl.BlockSpec((_D, _D), lambda i: (0, 0)),
            pl.BlockSpec((dt, _D), lambda i: (0, 0)),
            pl.BlockSpec((1, _D), lambda i: (0, 0)),
        ],
        out_specs=[
            pl.BlockSpec((_BM, _D), lambda i: (i, 0)),
            pl.BlockSpec((_BM, _D), lambda i: (i, 0)),
        ],
        out_shape=[
            jax.ShapeDtypeStruct((_N_MESH, _D), jnp.float32),
            jax.ShapeDtypeStruct((_N_MESH, _D), jnp.float32),
        ],
    )


_layer_a_first = _make_layer_a(_D_EDGE)
_layer_a_rest = _make_layer_a(_D)


def _layer_c_body(n, sp, g0, g1, w0a, w0b, b0, w1, b1, o_n, o_s):
    nn = n[...]
    s = sp[...] + g0[...] + g1[...]
    h = _silu(nn @ w0a[...] + s @ w0b[...] + b0[...])
    o_n[...] = nn + h @ w1[...] + b1[...]
    o_s[...] = s


_layer_c = pl.pallas_call(
    _layer_c_body,
    grid=(_N_MESH // _BM,),
    in_specs=[
        pl.BlockSpec((_BM, _D), lambda i: (i, 0)),
        pl.BlockSpec((_BM, _D), lambda i: (i, 0)),
        pl.BlockSpec((_BM, _D), lambda i: (i, 0)),
        pl.BlockSpec((_BM, _D), lambda i: (i, 0)),
        pl.BlockSpec((_D, _D), lambda i: (0, 0)),
        pl.BlockSpec((_D, _D), lambda i: (0, 0)),
        pl.BlockSpec((1, _D), lambda i: (0, 0)),
        pl.BlockSpec((_D, _D), lambda i: (0, 0)),
        pl.BlockSpec((1, _D), lambda i: (0, 0)),
    ],
    out_specs=[
        pl.BlockSpec((_BM, _D), lambda i: (i, 0)),
        pl.BlockSpec((_BM, _D), lambda i: (i, 0)),
    ],
    out_shape=[
        jax.ShapeDtypeStruct((_N_MESH, _D), jnp.float32),
        jax.ShapeDtypeStruct((_N_MESH, _D), jnp.float32),
    ],
)


def _make_sc_gather(n_rows, d, b_pad, krow, nbuf):
    chunk = krow * _CHUNK
    per_w = b_pad // _NW
    n_chunks = per_w // chunk
    n_grp = n_chunks // nbuf
    idx_rows = n_chunks + nbuf
    mesh = plsc.VectorSubcoreMesh(core_axis_name="c", subcore_axis_name="s")

    scratch = [pltpu.VMEM((idx_rows, chunk), jnp.int32)]
    scratch += [pltpu.VMEM((chunk, d), jnp.float32) for _ in range(nbuf)]
    scratch += [pltpu.SemaphoreType.DMA for _ in range(2 * nbuf)]

    @functools.partial(
        pl.kernel,
        out_type=jax.ShapeDtypeStruct((b_pad, d), jnp.float32),
        mesh=mesh,
        scratch_types=scratch,
    )
    def gather_kernel(table_hbm, idx_hbm, out_hbm, idx_v, *bufs):
        rows = bufs[:nbuf]
        gsems = bufs[nbuf:2 * nbuf]
        osems = bufs[2 * nbuf:]
        wid = lax.axis_index("s") * _NC + lax.axis_index("c")
        base = wid * per_w
        pltpu.sync_copy(idx_hbm.at[wid], idx_v)

        def idx_at(j):
            return idx_v.at[j]

        def grp_body(g, carry):
            j0 = g * nbuf
            ds = [pltpu.async_copy(table_hbm.at[idx_at(j0 + b)],
                                   rows[b], gsems[b])
                  for b in range(nbuf)]
            for dcp in ds:
                dcp.wait()
            ds = [pltpu.async_copy(
                      rows[b],
                      out_hbm.at[pl.ds(base + (j0 + b) * chunk, chunk)],
                      osems[b])
                  for b in range(nbuf)]
            for dcp in ds:
                dcp.wait()
            return carry

        lax.fori_loop(0, n_grp, grp_body, 0)

    return gather_kernel


def _make_sc_scatter_add(d, table_mode):
    nbuf = 2
    per_w = _E_PAD // _NW
    n_chunks = per_w // _CHUNK
    n_grp = n_chunks // nbuf
    idx_rows = n_chunks + nbuf
    rows_per_tile = _ACC_ROWS // _NS
    mesh = plsc.VectorSubcoreMesh(core_axis_name="c", subcore_axis_name="s")

    scratch = [
        pltpu.VMEM((idx_rows, _CHUNK), jnp.int32),
        pltpu.VMEM((idx_rows, _CHUNK), jnp.int32),
        pltpu.VMEM((16, d), jnp.float32),
        pltpu.VMEM_SHARED((_ACC_ROWS, d), jnp.float32),
        pltpu.SemaphoreType.DMA,
    ]
    scratch += [pltpu.VMEM((_CHUNK, d), jnp.float32) for _ in range(nbuf)]
    scratch += [pltpu.SemaphoreType.DMA for _ in range(2 * nbuf)]

    @functools.partial(
        pl.kernel,
        out_type=jax.ShapeDtypeStruct((_NC, _ACC_ROWS, d), jnp.float32),
        mesh=mesh,
        scratch_types=scratch,
    )
    def scatter_kernel(src_hbm, sidx_hbm, ridx_hbm, out_hbm,
                       sidx_v, ridx_v, zbuf, acc, zsem, *bufs):
        rows = bufs[:nbuf]
        gsems = bufs[nbuf:2 * nbuf]
        ssems = bufs[2 * nbuf:]
        cid = lax.axis_index("c")
        sid = lax.axis_index("s")
        wid = sid * _NC + cid
        row0 = sid * rows_per_tile

        zv = jnp.zeros((_LANES,), jnp.float32)

        def zb_body(r, carry):
            for c in range(d // _LANES):
                zbuf[r, pl.ds(c * _LANES, _LANES)] = zv
            return carry

        lax.fori_loop(0, 16, zb_body, 0)
        zds = [pltpu.async_copy(zbuf, acc.at[pl.ds(row0 + t * 16, 16)], zsem)
               for t in range(rows_per_tile // 16)]
        if table_mode:
            pltpu.sync_copy(sidx_hbm.at[wid], sidx_v)
        pltpu.sync_copy(ridx_hbm.at[wid], ridx_v)
        for dcp in zds:
            dcp.wait()
        plsc.subcore_barrier()

        def grp_body(g, carry):
            j0 = g * nbuf
            if table_mode:
                ds = [pltpu.async_copy(src_hbm.at[sidx_v.at[j0 + b]],
                                       rows[b], gsems[b])
                      for b in range(nbuf)]
            else:
                ds = [pltpu.async_copy(
                          src_hbm.at[pl.ds((wid * n_chunks + j0 + b) * _CHUNK,
                                           _CHUNK)],
                          rows[b], gsems[b])
                      for b in range(nbuf)]
            for dcp in ds:
                dcp.wait()
            ds = [pltpu.async_copy(rows[b], acc.at[ridx_v.at[j0 + b]],
                                   ssems[b], add=True)
                  for b in range(nbuf)]
            for dcp in ds:
                dcp.wait()
            return carry

        lax.fori_loop(0, n_grp, grp_body, 0)
        plsc.subcore_barrier()
        pltpu.sync_copy(acc.at[pl.ds(row0, rows_per_tile)],
                        out_hbm.at[cid, pl.ds(row0, rows_per_tile)])

    return scatter_kernel


_G2M_NBUF = 5
_M2G_NBUF = 5
_g2m_gather = _make_sc_gather(_N_GRID, _D, _G2M_PAD, krow=1, nbuf=_G2M_NBUF)
_m2g_gather = _make_sc_gather(_N_MESH, _D, _M2G_PAD, krow=1, nbuf=_M2G_NBUF)
_edge_scatter = _make_sc_scatter_add(_D, table_mode=True)
_init_scatter = _make_sc_scatter_add(_D, table_mode=False)
_SC_NBUF = 2


def _pack_idx(flat, b_pad, krow, nbuf):
    chunk = krow * _CHUNK
    arr = flat.reshape(_NW, b_pad // _NW)
    arr = jnp.concatenate(
        [arr, jnp.zeros((_NW, nbuf * chunk), jnp.int32)], axis=1)
    return arr.reshape(_NW, -1, chunk)


def kernel(grid_input, mesh_nodes, mesh_edges, mesh_senders, mesh_receivers,
           g2m_indices, g2m_weights, m2g_indices, m2g_weights,
           grid_enc_W0, grid_enc_b0, grid_enc_W1, grid_enc_b1,
           mesh_enc_W0, mesh_enc_b0, mesh_enc_W1, mesh_enc_b1,
           edge_W_first, edge_b_first, edge_W_rest, edge_b_rest,
           node_W0, node_b0, node_W1, node_b1,
           dec_W0, dec_b0, dec_W1, dec_b1):
    i32 = jnp.int32
    senders = mesh_senders.astype(i32)
    receivers = mesh_receivers.astype(i32)
    e_pad = _E_PAD - _N_EDGE
    sidx = _pack_idx(jnp.concatenate([senders, jnp.zeros((e_pad,), i32)]),
                     _E_PAD, 1, _SC_NBUF)
    ridx = _pack_idx(
        jnp.concatenate([receivers, jnp.full((e_pad,), _DUMMY_ROW, i32)]),
        _E_PAD, 1, _SC_NBUF)

    grid_feat = _grid_enc(grid_input, grid_enc_W0,
                          grid_enc_b0.reshape(1, _D),
                          grid_enc_W1, grid_enc_b1.reshape(1, _D))

    g2m_flat = _pack_idx(jnp.concatenate([
        g2m_indices.reshape(-1).astype(i32),
        jnp.zeros((_G2M_PAD - _N_MESH * _K,), i32)]),
        _G2M_PAD, 1, _G2M_NBUF)
    g2m_rows = _g2m_gather(grid_feat, g2m_flat)
    g2m_rows = g2m_rows[:_N_MESH * _K].reshape(_N_MESH, _K * _D)

    payload = jnp.concatenate([
        mesh_edges,
        jnp.ones((_N_EDGE, 1), jnp.float32),
        jnp.zeros((_N_EDGE, _D - _D_EDGE - 1), jnp.float32)], axis=1)
    payload = jnp.concatenate(
        [payload,
         jnp.zeros((e_pad + _SC_NBUF * _CHUNK, _D), jnp.float32)], axis=0)
    tparts = _init_scatter(payload, sidx, ridx)
    t_init, deg = _tdeg_combine(tparts[0, :_N_MESH], tparts[1, :_N_MESH])

    nodes = _mesh_enc(g2m_rows, g2m_weights, mesh_nodes,
                      mesh_enc_W0[:_D], mesh_enc_W0[_D:],
                      mesh_enc_b0.reshape(1, _D),
                      mesh_enc_W1, mesh_enc_b1.reshape(1, _D))

    t_prev = t_init
    for l in range(_L):
        if l == 0:
            ws, wr = edge_W_first[:_D], edge_W_first[_D:2 * _D]
            we, be = edge_W_first[2 * _D:], edge_b_first
            layer_a = _layer_a_first
        else:
            ew = edge_W_rest[l - 1]
            ws, wr, we = ew[:_D], ew[_D:2 * _D], ew[2 * _D:]
            be = edge_b_rest[l - 1]
            layer_a = _layer_a_rest
        p_s, s_pre = layer_a(nodes, t_prev, deg, ws, wr, we,
                             be.reshape(1, _D))
        gparts = _edge_scatter(p_s, sidx, ridx)
        nodes, t_prev = _layer_c(nodes, s_pre,
                                 gparts[0, :_N_MESH], gparts[1, :_N_MESH],
                                 node_W0[l, :_D], node_W0[l, _D:],
                                 node_b0[l].reshape(1, _D),
                                 node_W1[l], node_b1[l].reshape(1, _D))

    m2g_flat = _pack_idx(jnp.concatenate([
        m2g_indices.reshape(-1).astype(i32),
        jnp.zeros((_M2G_PAD - _N_GRID * _K,), i32)]),
        _M2G_PAD, 1, _M2G_NBUF)
    m2g_rows = _m2g_gather(nodes, m2g_flat)
    m2g_rows = m2g_rows[:_N_GRID * _K].reshape(_N_GRID, _K * _D)

    return _decoder(m2g_rows, m2g_weights, dec_W0,
                    dec_b0.reshape(1, _D), dec_W1,
                    dec_b1.reshape(1, _GRID_DIM))


# speedup vs baseline: 1.7837x; 1.0134x over previous
import functools

import jax
import jax.numpy as jnp
from jax import lax
from jax.experimental import pallas as pl
from jax.experimental.pallas import tpu as pltpu
from jax.experimental.pallas import tpu_sc as plsc

_N_GRID = 100000
_N_MESH = 10000
_N_EDGE = 160000
_D_EDGE = 16
_D = 128
_GRID_DIM = 2
_K = 4
_L = 6

_NC, _NS, _LANES = 2, 16, 16
_NW = _NC * _NS
_CHUNK = 128

_E_PAD = 163840
_ACC_ROWS = 10240
_DUMMY_ROW = _ACC_ROWS - 1
_G2M_PAD = 40960
_M2G_PAD = 409600

_BG = 2000
_BM = 2000


def _silu(x):
    return x * (1.0 / (1.0 + jnp.exp(-x)))


def _grid_enc_body(x, w0, b0, w1, b1, o):
    h = _silu(x[...] @ w0[...] + b0[...])
    o[...] = h @ w1[...] + b1[...]


_grid_enc = pl.pallas_call(
    _grid_enc_body,
    grid=(_N_GRID // _BG,),
    in_specs=[
        pl.BlockSpec((_BG, _GRID_DIM), lambda i: (i, 0)),
        pl.BlockSpec((_GRID_DIM, _D), lambda i: (0, 0)),
        pl.BlockSpec((1, _D), lambda i: (0, 0)),
        pl.BlockSpec((_D, _D), lambda i: (0, 0)),
        pl.BlockSpec((1, _D), lambda i: (0, 0)),
    ],
    out_specs=pl.BlockSpec((_BG, _D), lambda i: (i, 0)),
    out_shape=jax.ShapeDtypeStruct((_N_GRID, _D), jnp.float32),
)


def _tdeg_body(t0, t1, o_t, o_d):
    s = t0[...] + t1[...]
    o_t[...] = s[:, :_D_EDGE]
    o_d[...] = s[:, _D_EDGE:_D_EDGE + 1]


_tdeg_combine = pl.pallas_call(
    _tdeg_body,
    grid=(_N_MESH // _BM,),
    in_specs=[
        pl.BlockSpec((_BM, _D), lambda i: (i, 0)),
        pl.BlockSpec((_BM, _D), lambda i: (i, 0)),
    ],
    out_specs=[
        pl.BlockSpec((_BM, _D_EDGE), lambda i: (i, 0)),
        pl.BlockSpec((_BM, 1), lambda i: (i, 0)),
    ],
    out_shape=[
        jax.ShapeDtypeStruct((_N_MESH, _D_EDGE), jnp.float32),
        jax.ShapeDtypeStruct((_N_MESH, 1), jnp.float32),
    ],
)


def _pool_mlp_body(nb, w, extra, w0a, w0b, b0, w1, b1, o):
    x = nb[...]
    ww = w[...]
    pooled = x[:, 0 * _D:1 * _D] * ww[:, 0:1]
    pooled += x[:, 1 * _D:2 * _D] * ww[:, 1:2]
    pooled += x[:, 2 * _D:3 * _D] * ww[:, 2:3]
    pooled += x[:, 3 * _D:4 * _D] * ww[:, 3:4]
    h = _silu(pooled @ w0a[...] + extra[...] @ w0b[...] + b0[...])
    o[...] = h @ w1[...] + b1[...]


_mesh_enc = pl.pallas_call(
    _pool_mlp_body,
    grid=(_N_MESH // _BM,),
    in_specs=[
        pl.BlockSpec((_BM, _K * _D), lambda i: (i, 0)),
        pl.BlockSpec((_BM, _K), lambda i: (i, 0)),
        pl.BlockSpec((_BM, _D), lambda i: (i, 0)),
        pl.BlockSpec((_D, _D), lambda i: (0, 0)),
        pl.BlockSpec((_D, _D), lambda i: (0, 0)),
        pl.BlockSpec((1, _D), lambda i: (0, 0)),
        pl.BlockSpec((_D, _D), lambda i: (0, 0)),
        pl.BlockSpec((1, _D), lambda i: (0, 0)),
    ],
    out_specs=pl.BlockSpec((_BM, _D), lambda i: (i, 0)),
    out_shape=jax.ShapeDtypeStruct((_N_MESH, _D), jnp.float32),
)


def _dec_body(nb, w, w0, b0, w1, b1, o):
    x = nb[...]
    ww = w[...]
    pooled = x[:, 0 * _D:1 * _D] * ww[:, 0:1]
    pooled += x[:, 1 * _D:2 * _D] * ww[:, 1:2]
    pooled += x[:, 2 * _D:3 * _D] * ww[:, 2:3]
    pooled += x[:, 3 * _D:4 * _D] * ww[:, 3:4]
    h = _silu(pooled @ w0[...] + b0[...])
    o[...] = h @ w1[...] + b1[...]


_decoder = pl.pallas_call(
    _dec_body,
    grid=(_N_GRID // _BG,),
    in_specs=[
        pl.BlockSpec((_BG, _K * _D), lambda i: (i, 0)),
        pl.BlockSpec((_BG, _K), lambda i: (i, 0)),
        pl.BlockSpec((_D, _D), lambda i: (0, 0)),
        pl.BlockSpec((1, _D), lambda i: (0, 0)),
        pl.BlockSpec((_D, _GRID_DIM), lambda i: (0, 0)),
        pl.BlockSpec((1, _GRID_DIM), lambda i: (0, 0)),
    ],
    out_specs=pl.BlockSpec((_BG, _GRID_DIM), lambda i: (i, 0)),
    out_shape=jax.ShapeDtypeStruct((_N_GRID, _GRID_DIM), jnp.float32),
)


def _layer_a_body(n, t, dg, ws, wr, we, be, o_ps, o_sp):
    nn = n[...]
    o_ps[...] = nn @ ws[...]
    o_sp[...] = dg[...] * (nn @ wr[...] + be[...]) + t[...] @ we[...]


def _make_layer_a(dt):
    return pl.pallas_call(
        _layer_a_body,
        grid=(_N_MESH // _BM,),
        in_specs=[
            pl.BlockSpec((_BM, _D), lambda i: (i, 0)),
            pl.BlockSpec((_BM, dt), lambda i: (i, 0)),
            pl.BlockSpec((_BM, 1), lambda i: (i, 0)),
            pl.BlockSpec((_D, _D), lambda i: (0, 0)),
            pl.BlockSpec((_D, _D), lambda i: (0, 0)),
            pl.BlockSpec((dt, _D), lambda i: (0, 0)),
            pl.BlockSpec((1, _D), lambda i: (0, 0)),
        ],
        out_specs=[
            pl.BlockSpec((_BM, _D), lambda i: (i, 0)),
            pl.BlockSpec((_BM, _D), lambda i: (i, 0)),
        ],
        out_shape=[
            jax.ShapeDtypeStruct((_N_MESH, _D), jnp.float32),
            jax.ShapeDtypeStruct((_N_MESH, _D), jnp.float32),
        ],
    )


_layer_a_first = _make_layer_a(_D_EDGE)
_layer_a_rest = _make_layer_a(_D)


def _layer_c_body(n, sp, g0, g1, w0a, w0b, b0, w1, b1, o_n, o_s):
    nn = n[...]
    s = sp[...] + g0[...] + g1[...]
    h = _silu(nn @ w0a[...] + s @ w0b[...] + b0[...])
    o_n[...] = nn + h @ w1[...] + b1[...]
    o_s[...] = s


_layer_c = pl.pallas_call(
    _layer_c_body,
    grid=(_N_MESH // _BM,),
    in_specs=[
        pl.BlockSpec((_BM, _D), lambda i: (i, 0)),
        pl.BlockSpec((_BM, _D), lambda i: (i, 0)),
        pl.BlockSpec((_BM, _D), lambda i: (i, 0)),
        pl.BlockSpec((_BM, _D), lambda i: (i, 0)),
        pl.BlockSpec((_D, _D), lambda i: (0, 0)),
        pl.BlockSpec((_D, _D), lambda i: (0, 0)),
        pl.BlockSpec((1, _D), lambda i: (0, 0)),
        pl.BlockSpec((_D, _D), lambda i: (0, 0)),
        pl.BlockSpec((1, _D), lambda i: (0, 0)),
    ],
    out_specs=[
        pl.BlockSpec((_BM, _D), lambda i: (i, 0)),
        pl.BlockSpec((_BM, _D), lambda i: (i, 0)),
    ],
    out_shape=[
        jax.ShapeDtypeStruct((_N_MESH, _D), jnp.float32),
        jax.ShapeDtypeStruct((_N_MESH, _D), jnp.float32),
    ],
)


def _make_sc_gather(n_rows, d, b_pad, krow, nbuf):
    chunk = krow * _CHUNK
    per_w = b_pad // _NW
    n_chunks = per_w // chunk
    n_grp = n_chunks // nbuf
    idx_rows = n_chunks + nbuf
    mesh = plsc.VectorSubcoreMesh(core_axis_name="c", subcore_axis_name="s")

    scratch = [pltpu.VMEM((idx_rows, chunk), jnp.int32)]
    scratch += [pltpu.VMEM((chunk, d), jnp.float32) for _ in range(nbuf)]
    scratch += [pltpu.SemaphoreType.DMA for _ in range(2 * nbuf)]

    @functools.partial(
        pl.kernel,
        out_type=jax.ShapeDtypeStruct((b_pad, d), jnp.float32),
        mesh=mesh,
        scratch_types=scratch,
    )
    def gather_kernel(table_hbm, idx_hbm, out_hbm, idx_v, *bufs):
        rows = bufs[:nbuf]
        gsems = bufs[nbuf:2 * nbuf]
        osems = bufs[2 * nbuf:]
        wid = lax.axis_index("s") * _NC + lax.axis_index("c")
        base = wid * per_w
        pltpu.sync_copy(idx_hbm.at[wid], idx_v)

        def idx_at(j):
            return idx_v.at[j]

        def grp_body(g, carry):
            j0 = g * nbuf
            gds = [pltpu.async_copy(table_hbm.at[idx_at(j0 + b)],
                                    rows[b], gsems[b])
                   for b in range(nbuf)]
            ods = []
            for b in range(nbuf):
                gds[b].wait()
                ods.append(pltpu.async_copy(
                    rows[b],
                    out_hbm.at[pl.ds(base + (j0 + b) * chunk, chunk)],
                    osems[b]))
            for dcp in ods:
                dcp.wait()
            return carry

        lax.fori_loop(0, n_grp, grp_body, 0)

    return gather_kernel


def _make_sc_scatter_add(d, table_mode):
    nbuf = 2
    per_w = _E_PAD // _NW
    n_chunks = per_w // _CHUNK
    n_grp = n_chunks // nbuf
    idx_rows = n_chunks + nbuf
    rows_per_tile = _ACC_ROWS // _NS
    mesh = plsc.VectorSubcoreMesh(core_axis_name="c", subcore_axis_name="s")

    scratch = [
        pltpu.VMEM((idx_rows, _CHUNK), jnp.int32),
        pltpu.VMEM((idx_rows, _CHUNK), jnp.int32),
        pltpu.VMEM((16, d), jnp.float32),
        pltpu.VMEM_SHARED((_ACC_ROWS, d), jnp.float32),
        pltpu.SemaphoreType.DMA,
    ]
    scratch += [pltpu.VMEM((_CHUNK, d), jnp.float32) for _ in range(nbuf)]
    scratch += [pltpu.SemaphoreType.DMA for _ in range(2 * nbuf)]

    @functools.partial(
        pl.kernel,
        out_type=jax.ShapeDtypeStruct((_NC, _ACC_ROWS, d), jnp.float32),
        mesh=mesh,
        scratch_types=scratch,
    )
    def scatter_kernel(src_hbm, sidx_hbm, ridx_hbm, out_hbm,
                       sidx_v, ridx_v, zbuf, acc, zsem, *bufs):
        rows = bufs[:nbuf]
        gsems = bufs[nbuf:2 * nbuf]
        ssems = bufs[2 * nbuf:]
        cid = lax.axis_index("c")
        sid = lax.axis_index("s")
        wid = sid * _NC + cid
        row0 = sid * rows_per_tile

        zv = jnp.zeros((_LANES,), jnp.float32)

        def zb_body(r, carry):
            for c in range(d // _LANES):
                zbuf[r, pl.ds(c * _LANES, _LANES)] = zv
            return carry

        lax.fori_loop(0, 16, zb_body, 0)
        zds = [pltpu.async_copy(zbuf, acc.at[pl.ds(row0 + t * 16, 16)], zsem)
               for t in range(rows_per_tile // 16)]
        if table_mode:
            pltpu.sync_copy(sidx_hbm.at[wid], sidx_v)
        pltpu.sync_copy(ridx_hbm.at[wid], ridx_v)
        for dcp in zds:
            dcp.wait()
        plsc.subcore_barrier()

        def grp_body(g, carry):
            j0 = g * nbuf
            if table_mode:
                gds = [pltpu.async_copy(src_hbm.at[sidx_v.at[j0 + b]],
                                        rows[b], gsems[b])
                       for b in range(nbuf)]
            else:
                gds = [pltpu.async_copy(
                           src_hbm.at[pl.ds((wid * n_chunks + j0 + b) * _CHUNK,
                                            _CHUNK)],
                           rows[b], gsems[b])
                       for b in range(nbuf)]
            sds = []
            for b in range(nbuf):
                gds[b].wait()
                sds.append(pltpu.async_copy(rows[b], acc.at[ridx_v.at[j0 + b]],
                                            ssems[b], add=True))
            for dcp in sds:
                dcp.wait()
            return carry

        lax.fori_loop(0, n_grp, grp_body, 0)
        plsc.subcore_barrier()
        pltpu.sync_copy(acc.at[pl.ds(row0, rows_per_tile)],
                        out_hbm.at[cid, pl.ds(row0, rows_per_tile)])

    return scatter_kernel


_G2M_NBUF = 5
_M2G_NBUF = 5
_g2m_gather = _make_sc_gather(_N_GRID, _D, _G2M_PAD, krow=1, nbuf=_G2M_NBUF)
_m2g_gather = _make_sc_gather(_N_MESH, _D, _M2G_PAD, krow=1, nbuf=_M2G_NBUF)
_edge_scatter = _make_sc_scatter_add(_D, table_mode=True)
_init_scatter = _make_sc_scatter_add(_D, table_mode=False)
_SC_NBUF = 2


def _pack_idx(flat, b_pad, krow, nbuf):
    chunk = krow * _CHUNK
    arr = flat.reshape(_NW, b_pad // _NW)
    arr = jnp.concatenate(
        [arr, jnp.zeros((_NW, nbuf * chunk), jnp.int32)], axis=1)
    return arr.reshape(_NW, -1, chunk)


def kernel(grid_input, mesh_nodes, mesh_edges, mesh_senders, mesh_receivers,
           g2m_indices, g2m_weights, m2g_indices, m2g_weights,
           grid_enc_W0, grid_enc_b0, grid_enc_W1, grid_enc_b1,
           mesh_enc_W0, mesh_enc_b0, mesh_enc_W1, mesh_enc_b1,
           edge_W_first, edge_b_first, edge_W_rest, edge_b_rest,
           node_W0, node_b0, node_W1, node_b1,
           dec_W0, dec_b0, dec_W1, dec_b1):
    i32 = jnp.int32
    senders = mesh_senders.astype(i32)
    receivers = mesh_receivers.astype(i32)
    e_pad = _E_PAD - _N_EDGE
    sidx = _pack_idx(jnp.concatenate([senders, jnp.zeros((e_pad,), i32)]),
                     _E_PAD, 1, _SC_NBUF)
    ridx = _pack_idx(
        jnp.concatenate([receivers, jnp.full((e_pad,), _DUMMY_ROW, i32)]),
        _E_PAD, 1, _SC_NBUF)

    grid_feat = _grid_enc(grid_input, grid_enc_W0,
                          grid_enc_b0.reshape(1, _D),
                          grid_enc_W1, grid_enc_b1.reshape(1, _D))

    g2m_flat = _pack_idx(jnp.concatenate([
        g2m_indices.reshape(-1).astype(i32),
        jnp.zeros((_G2M_PAD - _N_MESH * _K,), i32)]),
        _G2M_PAD, 1, _G2M_NBUF)
    g2m_rows = _g2m_gather(grid_feat, g2m_flat)
    g2m_rows = g2m_rows[:_N_MESH * _K].reshape(_N_MESH, _K * _D)

    payload = jnp.concatenate([
        mesh_edges,
        jnp.ones((_N_EDGE, 1), jnp.float32),
        jnp.zeros((_N_EDGE, _D - _D_EDGE - 1), jnp.float32)], axis=1)
    payload = jnp.concatenate(
        [payload,
         jnp.zeros((e_pad + _SC_NBUF * _CHUNK, _D), jnp.float32)], axis=0)
    tparts = _init_scatter(payload, sidx, ridx)
    t_init, deg = _tdeg_combine(tparts[0, :_N_MESH], tparts[1, :_N_MESH])

    nodes = _mesh_enc(g2m_rows, g2m_weights, mesh_nodes,
                      mesh_enc_W0[:_D], mesh_enc_W0[_D:],
                      mesh_enc_b0.reshape(1, _D),
                      mesh_enc_W1, mesh_enc_b1.reshape(1, _D))

    t_prev = t_init
    for l in range(_L):
        if l == 0:
            ws, wr = edge_W_first[:_D], edge_W_first[_D:2 * _D]
            we, be = edge_W_first[2 * _D:], edge_b_first
            layer_a = _layer_a_first
        else:
            ew = edge_W_rest[l - 1]
            ws, wr, we = ew[:_D], ew[_D:2 * _D], ew[2 * _D:]
            be = edge_b_rest[l - 1]
            layer_a = _layer_a_rest
        p_s, s_pre = layer_a(nodes, t_prev, deg, ws, wr, we,
                             be.reshape(1, _D))
        gparts = _edge_scatter(p_s, sidx, ridx)
        nodes, t_prev = _layer_c(nodes, s_pre,
                                 gparts[0, :_N_MESH], gparts[1, :_N_MESH],
                                 node_W0[l, :_D], node_W0[l, _D:],
                                 node_b0[l].reshape(1, _D),
                                 node_W1[l], node_b1[l].reshape(1, _D))

    m2g_flat = _pack_idx(jnp.concatenate([
        m2g_indices.reshape(-1).astype(i32),
        jnp.zeros((_M2G_PAD - _N_GRID * _K,), i32)]),
        _M2G_PAD, 1, _M2G_NBUF)
    m2g_rows = _m2g_gather(nodes, m2g_flat)
    m2g_rows = m2g_rows[:_N_GRID * _K].reshape(_N_GRID, _K * _D)

    return _decoder(m2g_rows, m2g_weights, dec_W0,
                    dec_b0.reshape(1, _D), dec_W1,
                    dec_b1.reshape(1, _GRID_DIM))
